# baseline (speedup 1.0000x reference)
import functools

import jax
import jax.numpy as jnp
from jax import lax
from jax.experimental import pallas as pl
from jax.experimental.pallas import tpu as pltpu
from jax.experimental.pallas import tpu_sc as plsc

N = 10000
E = 320000
D = 128
DH = D // 2

NC = 2
NS = 16
NW = NC * NS
L = 16
B = 128
CHP = 2560
E_PAD = CHP * B
NCHUNK = CHP // NS
N_PAD = 10240
ROWS_PT = N_PAD // NS
NBUF = 2
NGRP = NCHUNK // NBUF
PACK = 1 << 14

_mesh = plsc.VectorSubcoreMesh(
    core_axis_name="c", subcore_axis_name="s", num_cores=NC, num_subcores=NS
)


@functools.partial(
    pl.kernel,
    out_type=jax.ShapeDtypeStruct((NC, 1, N_PAD), jnp.float32),
    mesh=_mesh,
    scratch_types=[
        pltpu.VMEM((CHP // NW, B), jnp.int32),
        pltpu.VMEM((B,), jnp.float32),
        pltpu.VMEM_SHARED((N_PAD,), jnp.float32),
        pltpu.SemaphoreType.DMA,
    ],
)
def _sc_degree(dst_hbm, ones_hbm, zeros_hbm, out_hbm, dst_v, ones_v, deg_sh, sem):
    c = lax.axis_index("c")
    s = lax.axis_index("s")
    w = c * NS + s
    nch = CHP // NW
    pltpu.sync_copy(
        zeros_hbm.at[pl.ds(s * ROWS_PT, ROWS_PT)],
        deg_sh.at[pl.ds(s * ROWS_PT, ROWS_PT)],
    )
    pltpu.sync_copy(dst_hbm.at[pl.ds(w * nch, nch)], dst_v)
    pltpu.sync_copy(ones_hbm, ones_v)
    plsc.subcore_barrier()

    def fire(j, carry):
        pltpu.async_copy(ones_v, deg_sh.at[dst_v.at[j]], sem, add=True)
        return carry

    lax.fori_loop(0, nch, fire, 0)

    def drain(j, carry):
        pltpu.make_async_copy(ones_v, deg_sh.at[dst_v.at[j]], sem).wait()
        return carry

    lax.fori_loop(0, nch, drain, 0)
    plsc.subcore_barrier()
    pltpu.sync_copy(
        deg_sh.at[pl.ds(s * ROWS_PT, ROWS_PT)],
        out_hbm.at[c, 0, pl.ds(s * ROWS_PT, ROWS_PT)],
    )


@functools.partial(
    pl.kernel,
    out_type=jax.ShapeDtypeStruct((NC, N_PAD, DH), jnp.float32),
    mesh=_mesh,
    scratch_types=[
        pltpu.VMEM((NCHUNK, B), jnp.int32),
        pltpu.VMEM((NBUF, B), jnp.int32),
        pltpu.VMEM((NBUF, B), jnp.int32),
        pltpu.VMEM((NBUF, B, DH), jnp.float32),
        pltpu.VMEM_SHARED((N_PAD, DH), jnp.float32),
        pltpu.VMEM_SHARED((N_PAD, DH), jnp.float32),
        pltpu.SemaphoreType.DMA((NBUF,)),
        pltpu.SemaphoreType.DMA((NBUF,)),
    ],
    compiler_params=pltpu.CompilerParams(use_tc_tiling_on_sc=False),
)
def _sc_scatter(hp2_hbm, pk_hbm, zeros_hbm, out_hbm,
                pk_v, sidx_v, didx_v, rows_v, tab_sh, agg_sh, gsem, ssem):
    c = lax.axis_index("c")
    s = lax.axis_index("s")
    pltpu.sync_copy(
        zeros_hbm.at[pl.ds(s * ROWS_PT, ROWS_PT)],
        agg_sh.at[pl.ds(s * ROWS_PT, ROWS_PT)],
    )
    pltpu.sync_copy(
        hp2_hbm.at[c, pl.ds(s * ROWS_PT, ROWS_PT)],
        tab_sh.at[pl.ds(s * ROWS_PT, ROWS_PT)],
    )
    pltpu.sync_copy(pk_hbm.at[pl.ds(s * NCHUNK, NCHUNK)], pk_v)
    plsc.subcore_barrier()

    def unpack(j, slot):
        for k in range(B // L):
            v = pk_v[j, pl.ds(k * L, L)]
            sidx_v[slot, pl.ds(k * L, L)] = v & (PACK - 1)
            didx_v[slot, pl.ds(k * L, L)] = v >> 14

    def fire_gather(ch, b):
        hbm_pick = (ch % 4) == 3

        @pl.when(jnp.logical_not(hbm_pick))
        def _():
            pltpu.async_copy(tab_sh.at[sidx_v.at[b]], rows_v.at[b], gsem.at[b])

        @pl.when(hbm_pick)
        def _():
            pltpu.async_copy(
                hp2_hbm.at[c].at[sidx_v.at[b]], rows_v.at[b], gsem.at[b]
            )

    for b in range(NBUF):
        unpack(b, b)
        fire_gather(jnp.int32(b), b)

    def group(g, carry):
        base = g * NBUF
        for b in range(NBUF):
            pltpu.make_async_copy(
                tab_sh.at[sidx_v.at[b]], rows_v.at[b], gsem.at[b]
            ).wait()
            pltpu.async_copy(
                rows_v.at[b], agg_sh.at[didx_v.at[b]], ssem.at[b], add=True
            )
        for b in range(NBUF):
            pltpu.make_async_copy(
                rows_v.at[b], agg_sh.at[didx_v.at[b]], ssem.at[b]
            ).wait()
            nxt = jnp.minimum(base + NBUF + b, NCHUNK - 1)

            @pl.when(g + 1 < NGRP)
            def _():
                unpack(nxt, b)
                fire_gather(nxt, b)

        return carry

    lax.fori_loop(0, NGRP, group, 0)
    plsc.subcore_barrier()
    pltpu.sync_copy(
        agg_sh.at[pl.ds(s * ROWS_PT, ROWS_PT)],
        out_hbm.at[c, pl.ds(s * ROWS_PT, ROWS_PT)],
    )


def _mm_body(x_ref, w_ref, h_ref):
    h_ref[...] = jnp.dot(x_ref[...], w_ref[...], preferred_element_type=jnp.float32)


_mm = pl.pallas_call(
    _mm_body,
    out_shape=jax.ShapeDtypeStruct((N, D), jnp.float32),
)


RBS = 1000


def _scale_body(h_ref, degT_ref, hp2_ref, dinv_ref):
    deg = degT_ref[:, 0:1] + degT_ref[:, 1:2] + 1.0
    dinv = lax.rsqrt(deg)
    hp = h_ref[...] * dinv
    hp2_ref[0] = hp[:, :DH]
    hp2_ref[1] = hp[:, DH:]
    dinv_ref[...] = dinv


_scale = pl.pallas_call(
    _scale_body,
    grid=(N // RBS,),
    in_specs=[
        pl.BlockSpec((RBS, D), lambda i: (i, 0)),
        pl.BlockSpec((RBS, 2), lambda i: (i, 0)),
    ],
    out_specs=(
        pl.BlockSpec((NC, RBS, DH), lambda i: (0, i, 0)),
        pl.BlockSpec((RBS, 1), lambda i: (i, 0)),
    ),
    out_shape=(
        jax.ShapeDtypeStruct((NC, N_PAD, DH), jnp.float32),
        jax.ShapeDtypeStruct((N, 1), jnp.float32),
    ),
)

RB = 1000


def _epilogue_body(sp_ref, h_ref, dinv_ref, x_ref, b_ref, g_ref, be_ref, out_ref):
    ssum = jnp.concatenate([sp_ref[0], sp_ref[1]], axis=-1)
    dinv = dinv_ref[...]
    agg = (ssum + h_ref[...] * dinv) * dinv + b_ref[...]
    mu = jnp.mean(agg, axis=-1, keepdims=True)
    cen = agg - mu
    var = jnp.mean(cen * cen, axis=-1, keepdims=True)
    ln = cen * lax.rsqrt(var + 1e-5) * g_ref[...] + be_ref[...]
    o = ln + x_ref[...]
    out_ref[...] = o * (1.0 / (1.0 + jnp.exp(-o)))


_epilogue = pl.pallas_call(
    _epilogue_body,
    grid=(N // RB,),
    in_specs=[
        pl.BlockSpec((NC, RB, DH), lambda i: (0, i, 0)),
        pl.BlockSpec((RB, D), lambda i: (i, 0)),
        pl.BlockSpec((RB, 1), lambda i: (i, 0)),
        pl.BlockSpec((RB, D), lambda i: (i, 0)),
        pl.BlockSpec((1, D), lambda i: (0, 0)),
        pl.BlockSpec((1, D), lambda i: (0, 0)),
        pl.BlockSpec((1, D), lambda i: (0, 0)),
    ],
    out_specs=pl.BlockSpec((RB, D), lambda i: (i, 0)),
    out_shape=jax.ShapeDtypeStruct((N, D), jnp.float32),
)


def kernel(x, edge_index, W, b, gamma, beta):
    npad = E_PAD - E
    src_pad = jnp.concatenate([edge_index[0], jnp.zeros((npad,), jnp.int32)])
    dst_fill = N + (jnp.arange(npad, dtype=jnp.int32) % (N_PAD - N))
    dst_pad = jnp.concatenate([edge_index[1], dst_fill])
    pk2d = (src_pad + dst_pad * PACK).reshape(CHP, B)
    dst2d = dst_pad.reshape(CHP, B)
    ones_row = jnp.ones((B,), jnp.float32)
    zeros1 = jnp.zeros((N_PAD,), jnp.float32)
    zeros2 = jnp.zeros((N_PAD, DH), jnp.float32)

    h = _mm(x, W)
    degp = _sc_degree(dst2d, ones_row, zeros1)
    degT = jnp.transpose(degp.reshape(NC, N_PAD))[:N]
    hp2, dinv = _scale(h, degT)
    sp = _sc_scatter(hp2, pk2d, zeros2)
    return _epilogue(
        sp, h, dinv, x,
        b.reshape(1, D), gamma.reshape(1, D), beta.reshape(1, D),
    )


# speedup vs baseline: 1.1686x; 1.1686x over previous
import functools

import jax
import jax.numpy as jnp
from jax import lax
from jax.experimental import pallas as pl
from jax.experimental.pallas import tpu as pltpu
from jax.experimental.pallas import tpu_sc as plsc

N = 10000
E = 320000
D = 128
DH = D // 2

NC = 2
NS = 16
NW = NC * NS
L = 16
B = 128
CHP = 2560
E_PAD = CHP * B
NCHUNK = CHP // NS
N_PAD = 10240
ROWS_PT = N_PAD // NS
NBUF = 2
NGRP = NCHUNK // NBUF
PACK = 1 << 14

_mesh = plsc.VectorSubcoreMesh(
    core_axis_name="c", subcore_axis_name="s", num_cores=NC, num_subcores=NS
)


@functools.partial(
    pl.kernel,
    out_type=jax.ShapeDtypeStruct((NC, 1, N_PAD), jnp.float32),
    mesh=_mesh,
    scratch_types=[
        pltpu.VMEM((CHP // NW, B), jnp.int32),
        pltpu.VMEM((B,), jnp.float32),
        pltpu.VMEM_SHARED((N_PAD,), jnp.float32),
        pltpu.SemaphoreType.DMA,
    ],
)
def _sc_degree(dst_hbm, ones_hbm, zeros_hbm, out_hbm, dst_v, ones_v, deg_sh, sem):
    c = lax.axis_index("c")
    s = lax.axis_index("s")
    w = c * NS + s
    nch = CHP // NW
    pltpu.sync_copy(
        zeros_hbm.at[pl.ds(s * ROWS_PT, ROWS_PT)],
        deg_sh.at[pl.ds(s * ROWS_PT, ROWS_PT)],
    )
    pltpu.sync_copy(dst_hbm.at[pl.ds(w * nch, nch)], dst_v)
    pltpu.sync_copy(ones_hbm, ones_v)
    plsc.subcore_barrier()

    def fire(j, carry):
        pltpu.async_copy(ones_v, deg_sh.at[dst_v.at[j]], sem, add=True)
        return carry

    lax.fori_loop(0, nch, fire, 0)

    def drain(j, carry):
        pltpu.make_async_copy(ones_v, deg_sh.at[dst_v.at[j]], sem).wait()
        return carry

    lax.fori_loop(0, nch, drain, 0)
    plsc.subcore_barrier()
    pltpu.sync_copy(
        deg_sh.at[pl.ds(s * ROWS_PT, ROWS_PT)],
        out_hbm.at[c, 0, pl.ds(s * ROWS_PT, ROWS_PT)],
    )


@functools.partial(
    pl.kernel,
    out_type=jax.ShapeDtypeStruct((NC, N_PAD, DH), jnp.float32),
    mesh=_mesh,
    scratch_types=[
        pltpu.VMEM((NCHUNK, B), jnp.int32),
        pltpu.VMEM((NBUF, B), jnp.int32),
        pltpu.VMEM((NBUF, B), jnp.int32),
        pltpu.VMEM((NBUF, B, DH), jnp.float32),
        pltpu.VMEM_SHARED((N_PAD, DH), jnp.float32),
        pltpu.VMEM_SHARED((N_PAD, DH), jnp.float32),
        pltpu.SemaphoreType.DMA((NBUF,)),
        pltpu.SemaphoreType.DMA((NBUF,)),
    ],
    compiler_params=pltpu.CompilerParams(use_tc_tiling_on_sc=False),
)
def _sc_scatter(hp2_hbm, pk_hbm, zeros_hbm, out_hbm,
                pk_v, sidx_v, didx_v, rows_v, tab_sh, agg_sh, gsem, ssem):
    c = lax.axis_index("c")
    s = lax.axis_index("s")
    pltpu.sync_copy(
        zeros_hbm.at[pl.ds(s * ROWS_PT, ROWS_PT)],
        agg_sh.at[pl.ds(s * ROWS_PT, ROWS_PT)],
    )
    pltpu.sync_copy(
        hp2_hbm.at[c, pl.ds(s * ROWS_PT, ROWS_PT)],
        tab_sh.at[pl.ds(s * ROWS_PT, ROWS_PT)],
    )
    pltpu.sync_copy(pk_hbm.at[pl.ds(s * NCHUNK, NCHUNK)], pk_v)
    plsc.subcore_barrier()

    def unpack(j, slot):
        for k in range(B // L):
            v = pk_v[j, pl.ds(k * L, L)]
            sidx_v[slot, pl.ds(k * L, L)] = v & (PACK - 1)
            didx_v[slot, pl.ds(k * L, L)] = v >> 14

    for b in range(NBUF):
        unpack(b, b)
        pltpu.async_copy(tab_sh.at[sidx_v.at[b]], rows_v.at[b], gsem.at[b])

    def group(g, carry):
        base = g * NBUF
        for b in range(NBUF):
            pltpu.make_async_copy(
                tab_sh.at[sidx_v.at[b]], rows_v.at[b], gsem.at[b]
            ).wait()
            pltpu.async_copy(
                rows_v.at[b], agg_sh.at[didx_v.at[b]], ssem.at[b], add=True
            )
        for b in range(NBUF):
            pltpu.make_async_copy(
                rows_v.at[b], agg_sh.at[didx_v.at[b]], ssem.at[b]
            ).wait()
            nxt = jnp.minimum(base + NBUF + b, NCHUNK - 1)

            @pl.when(g + 1 < NGRP)
            def _():
                unpack(nxt, b)
                pltpu.async_copy(tab_sh.at[sidx_v.at[b]], rows_v.at[b], gsem.at[b])

        return carry

    lax.fori_loop(0, NGRP, group, 0)
    plsc.subcore_barrier()
    pltpu.sync_copy(
        agg_sh.at[pl.ds(s * ROWS_PT, ROWS_PT)],
        out_hbm.at[c, pl.ds(s * ROWS_PT, ROWS_PT)],
    )


def _mm_body(x_ref, w_ref, h_ref):
    h_ref[...] = jnp.dot(x_ref[...], w_ref[...], preferred_element_type=jnp.float32)


_mm = pl.pallas_call(
    _mm_body,
    out_shape=jax.ShapeDtypeStruct((N, D), jnp.float32),
)


RBS = 1000


def _scale_body(h_ref, degT_ref, hp2_ref, dinv_ref):
    deg = degT_ref[:, 0:1] + degT_ref[:, 1:2] + 1.0
    dinv = lax.rsqrt(deg)
    hp = h_ref[...] * dinv
    hp2_ref[0] = hp[:, :DH]
    hp2_ref[1] = hp[:, DH:]
    dinv_ref[...] = dinv


_scale = pl.pallas_call(
    _scale_body,
    grid=(N // RBS,),
    in_specs=[
        pl.BlockSpec((RBS, D), lambda i: (i, 0)),
        pl.BlockSpec((RBS, 2), lambda i: (i, 0)),
    ],
    out_specs=(
        pl.BlockSpec((NC, RBS, DH), lambda i: (0, i, 0)),
        pl.BlockSpec((RBS, 1), lambda i: (i, 0)),
    ),
    out_shape=(
        jax.ShapeDtypeStruct((NC, N_PAD, DH), jnp.float32),
        jax.ShapeDtypeStruct((N, 1), jnp.float32),
    ),
)

RB = 1000


def _epilogue_body(sp_ref, h_ref, dinv_ref, x_ref, b_ref, g_ref, be_ref, out_ref):
    ssum = jnp.concatenate([sp_ref[0], sp_ref[1]], axis=-1)
    dinv = dinv_ref[...]
    agg = (ssum + h_ref[...] * dinv) * dinv + b_ref[...]
    mu = jnp.mean(agg, axis=-1, keepdims=True)
    cen = agg - mu
    var = jnp.mean(cen * cen, axis=-1, keepdims=True)
    ln = cen * lax.rsqrt(var + 1e-5) * g_ref[...] + be_ref[...]
    o = ln + x_ref[...]
    out_ref[...] = o * (1.0 / (1.0 + jnp.exp(-o)))


_epilogue = pl.pallas_call(
    _epilogue_body,
    grid=(N // RB,),
    in_specs=[
        pl.BlockSpec((NC, RB, DH), lambda i: (0, i, 0)),
        pl.BlockSpec((RB, D), lambda i: (i, 0)),
        pl.BlockSpec((RB, 1), lambda i: (i, 0)),
        pl.BlockSpec((RB, D), lambda i: (i, 0)),
        pl.BlockSpec((1, D), lambda i: (0, 0)),
        pl.BlockSpec((1, D), lambda i: (0, 0)),
        pl.BlockSpec((1, D), lambda i: (0, 0)),
    ],
    out_specs=pl.BlockSpec((RB, D), lambda i: (i, 0)),
    out_shape=jax.ShapeDtypeStruct((N, D), jnp.float32),
)


def kernel(x, edge_index, W, b, gamma, beta):
    npad = E_PAD - E
    src_pad = jnp.concatenate([edge_index[0], jnp.zeros((npad,), jnp.int32)])
    dst_fill = N + (jnp.arange(npad, dtype=jnp.int32) % (N_PAD - N))
    dst_pad = jnp.concatenate([edge_index[1], dst_fill])
    pk2d = (src_pad + dst_pad * PACK).reshape(CHP, B)
    dst2d = dst_pad.reshape(CHP, B)
    ones_row = jnp.ones((B,), jnp.float32)
    zeros1 = jnp.zeros((N_PAD,), jnp.float32)
    zeros2 = jnp.zeros((N_PAD, DH), jnp.float32)

    h = _mm(x, W)
    degp = _sc_degree(dst2d, ones_row, zeros1)
    degT = jnp.transpose(degp.reshape(NC, N_PAD))[:N]
    hp2, dinv = _scale(h, degT)
    sp = _sc_scatter(hp2, pk2d, zeros2)
    return _epilogue(
        sp, h, dinv, x,
        b.reshape(1, D), gamma.reshape(1, D), beta.reshape(1, D),
    )
